# Initial kernel scaffold; baseline (speedup 1.0000x reference)
import jax
import jax.numpy as jnp
from jax.experimental import pallas as pl


def kernel(eps, loc, log_scale, weight_scores, mode_ind, num_samples):
    raise NotImplementedError("write your pallas kernel here")


# speedup vs baseline: 1.2387x; 1.2387x over previous
import functools
import math

import jax
import jax.numpy as jnp
from jax import lax
from jax.experimental import pallas as pl
from jax.experimental.pallas import tpu as pltpu
from jax.experimental.pallas import tpu_sc as plsc

N_MODES = 64
DIM = 64
NC = 2
NS = 16
NW = NC * NS
L = 16
CH = 256


def _sc_kernel(n):
    mesh = plsc.VectorSubcoreMesh(core_axis_name="c", subcore_axis_name="s")
    per_w = n // NW
    nch = per_w // CH

    @functools.partial(
        pl.kernel,
        mesh=mesh,
        compiler_params=pltpu.CompilerParams(needs_layout_passes=False),
        out_type=[
            jax.ShapeDtypeStruct((n * DIM,), jnp.float32),
            jax.ShapeDtypeStruct((n,), jnp.float32),
        ],
        scratch_types=[
            pltpu.VMEM((N_MODES * DIM,), jnp.float32),
            pltpu.VMEM((N_MODES * DIM,), jnp.float32),
            pltpu.VMEM((L,), jnp.float32),
            pltpu.VMEM((CH,), jnp.int32),
            pltpu.VMEM((CH * DIM,), jnp.float32),
            pltpu.VMEM((CH * DIM,), jnp.float32),
            pltpu.VMEM((CH,), jnp.float32),
        ],
    )
    def k(ls_hbm, loc_hbm, c_hbm, idx_hbm, eps_hbm,
          z_hbm, lp_hbm,
          scale_v, loc_v, c_v, idx_v, eps_v, z_v, lp_v):
        wid = lax.axis_index("s") * NC + lax.axis_index("c")

        pltpu.sync_copy(ls_hbm, scale_v)
        pltpu.sync_copy(loc_hbm, loc_v)
        pltpu.sync_copy(c_hbm, c_v)

        def exp_body(i, _):
            scale_v[pl.ds(i * L, L)] = jnp.exp(scale_v[pl.ds(i * L, L)])
            return 0
        lax.fori_loop(0, (N_MODES * DIM) // L, exp_body, 0)

        cvec = c_v[...]
        lane_off = lax.iota(jnp.int32, L) * DIM

        def chunk_body(ci, _):
            sbase = wid * per_w + ci * CH
            pltpu.sync_copy(idx_hbm.at[pl.ds(sbase, CH)], idx_v)
            pltpu.sync_copy(eps_hbm.at[pl.ds(sbase * DIM, CH * DIM)], eps_v)

            def group_body(g, _):
                mode_vec = idx_v[pl.ds(g * L, L)]
                trow = mode_vec * DIM
                ebase = lane_off + g * (L * DIM)
                acc = jnp.zeros((L,), jnp.float32)
                for d in range(DIM):
                    ei = ebase + d
                    ev = plsc.load_gather(eps_v, [ei])
                    sv = plsc.load_gather(scale_v, [trow + d])
                    lv = plsc.load_gather(loc_v, [trow + d])
                    plsc.store_scatter(z_v, [ei], ev * sv + lv)
                    acc = acc + ev * ev
                lp_v[pl.ds(g * L, L)] = cvec - 0.5 * acc
                return 0
            lax.fori_loop(0, CH // L, group_body, 0)

            pltpu.sync_copy(z_v, z_hbm.at[pl.ds(sbase * DIM, CH * DIM)])
            pltpu.sync_copy(lp_v, lp_hbm.at[pl.ds(sbase, CH)])
            return 0
        lax.fori_loop(0, nch, chunk_body, 0)

    return k


def kernel(eps, loc, log_scale, weight_scores, mode_ind, num_samples):
    n = eps.shape[0]
    log_w = jax.nn.log_softmax(weight_scores, axis=1)
    per_mode = log_w - jnp.sum(log_scale, axis=2)
    c = (-0.5 * DIM * math.log(2.0 * math.pi)
         + jax.scipy.special.logsumexp(per_mode, axis=1))
    c_arr = jnp.broadcast_to(c.astype(jnp.float32), (L,))

    z_flat, log_p = _sc_kernel(n)(
        log_scale.reshape(-1).astype(jnp.float32),
        loc.reshape(-1).astype(jnp.float32),
        c_arr,
        mode_ind.astype(jnp.int32),
        eps.reshape(-1),
    )
    return z_flat.reshape(n, DIM), log_p


# speedup vs baseline: 3.6723x; 2.9647x over previous
import functools
import math

import jax
import jax.numpy as jnp
from jax import lax
from jax.experimental import pallas as pl
from jax.experimental.pallas import tpu as pltpu
from jax.experimental.pallas import tpu_sc as plsc

N_MODES = 64
DIM = 64
NC = 2
NS = 16
NW = NC * NS
L = 16
CH = 256
UNROLL = 4


def _sc_kernel(n):
    mesh = plsc.VectorSubcoreMesh(core_axis_name="c", subcore_axis_name="s")
    per_w = n // NW
    nch = per_w // CH

    @functools.partial(
        pl.kernel,
        mesh=mesh,
        compiler_params=pltpu.CompilerParams(needs_layout_passes=False),
        out_type=[
            jax.ShapeDtypeStruct((n, DIM), jnp.float32),
            jax.ShapeDtypeStruct((n,), jnp.float32),
        ],
        scratch_types=[
            pltpu.VMEM((N_MODES * DIM,), jnp.float32),
            pltpu.VMEM((N_MODES * DIM,), jnp.float32),
            pltpu.VMEM((L,), jnp.float32),
            pltpu.VMEM((CH,), jnp.int32),
            pltpu.VMEM((CH, DIM), jnp.float32),
            pltpu.VMEM((CH, DIM), jnp.float32),
            pltpu.VMEM((CH,), jnp.float32),
        ],
    )
    def k(ls_hbm, loc_hbm, c_hbm, idx_hbm, eps_hbm,
          z_hbm, lp_hbm,
          scale_v, loc_v, c_v, idx_v, eps_v, z_v, lp_v):
        wid = lax.axis_index("s") * NC + lax.axis_index("c")

        pltpu.sync_copy(ls_hbm, scale_v)
        pltpu.sync_copy(loc_hbm, loc_v)
        pltpu.sync_copy(c_hbm, c_v)

        def exp_body(i, _):
            scale_v[pl.ds(i * L, L)] = jnp.exp(scale_v[pl.ds(i * L, L)])
            return 0
        lax.fori_loop(0, (N_MODES * DIM) // L, exp_body, 0)

        cvec = c_v[...]
        lane_iota = lax.iota(jnp.int32, L)

        def chunk_body(ci, _):
            sbase = wid * per_w + ci * CH
            pltpu.sync_copy(idx_hbm.at[pl.ds(sbase, CH)], idx_v)
            pltpu.sync_copy(eps_hbm.at[pl.ds(sbase, CH), :], eps_v)

            def group_body(g, _):
                trow_vec = idx_v[pl.ds(g * L, L)] * DIM
                sums = jnp.zeros((L,), jnp.float32)
                for u in range(L):
                    s = g * L + u
                    trow = trow_vec[u]
                    acc = jnp.zeros((L,), jnp.float32)
                    for b in range(DIM // L):
                        ev = eps_v[s, pl.ds(b * L, L)]
                        sv = scale_v[pl.ds(trow + b * L, L)]
                        lv = loc_v[pl.ds(trow + b * L, L)]
                        z_v[s, pl.ds(b * L, L)] = ev * sv + lv
                        acc = acc + ev * ev
                    sums = jnp.where(lane_iota == u, jnp.sum(acc), sums)
                lp_v[pl.ds(g * L, L)] = cvec - 0.5 * sums
                return 0
            lax.fori_loop(0, CH // L, group_body, 0)

            pltpu.sync_copy(z_v, z_hbm.at[pl.ds(sbase, CH), :])
            pltpu.sync_copy(lp_v, lp_hbm.at[pl.ds(sbase, CH)])
            return 0
        lax.fori_loop(0, nch, chunk_body, 0)

    return k


def kernel(eps, loc, log_scale, weight_scores, mode_ind, num_samples):
    n = eps.shape[0]
    log_w = jax.nn.log_softmax(weight_scores, axis=1)
    per_mode = log_w - jnp.sum(log_scale, axis=2)
    c = (-0.5 * DIM * math.log(2.0 * math.pi)
         + jax.scipy.special.logsumexp(per_mode, axis=1))
    c_arr = jnp.broadcast_to(c.astype(jnp.float32), (L,))

    z, log_p = _sc_kernel(n)(
        log_scale.reshape(-1).astype(jnp.float32),
        loc.reshape(-1).astype(jnp.float32),
        c_arr,
        mode_ind.astype(jnp.int32),
        eps,
    )
    return z, log_p


# speedup vs baseline: 3.6735x; 1.0003x over previous
import functools
import math

import jax
import jax.numpy as jnp
from jax import lax
from jax.experimental import pallas as pl
from jax.experimental.pallas import tpu as pltpu
from jax.experimental.pallas import tpu_sc as plsc

N_MODES = 64
DIM = 64
NC = 2
NS = 16
NW = NC * NS
L = 16
CH = 256
UNROLL = 4


def _sc_kernel(n):
    mesh = plsc.VectorSubcoreMesh(core_axis_name="c", subcore_axis_name="s")
    per_w = n // NW
    nch = per_w // CH

    @functools.partial(
        pl.kernel,
        mesh=mesh,
        compiler_params=pltpu.CompilerParams(
            needs_layout_passes=False, use_tc_tiling_on_sc=True),
        out_type=[
            jax.ShapeDtypeStruct((n, DIM), jnp.float32),
            jax.ShapeDtypeStruct((n,), jnp.float32),
        ],
        scratch_types=[
            pltpu.VMEM((N_MODES * DIM,), jnp.float32),
            pltpu.VMEM((N_MODES * DIM,), jnp.float32),
            pltpu.VMEM((L,), jnp.float32),
            pltpu.VMEM((CH,), jnp.int32),
            pltpu.VMEM((CH, DIM), jnp.float32),
            pltpu.VMEM((CH, DIM), jnp.float32),
            pltpu.VMEM((CH,), jnp.float32),
        ],
    )
    def k(ls_hbm, loc_hbm, c_hbm, idx_hbm, eps_hbm,
          z_hbm, lp_hbm,
          scale_v, loc_v, c_v, idx_v, eps_v, z_v, lp_v):
        wid = lax.axis_index("s") * NC + lax.axis_index("c")

        pltpu.sync_copy(ls_hbm, scale_v)
        pltpu.sync_copy(loc_hbm, loc_v)
        pltpu.sync_copy(c_hbm, c_v)

        def exp_body(i, _):
            scale_v[pl.ds(i * L, L)] = jnp.exp(scale_v[pl.ds(i * L, L)])
            return 0
        lax.fori_loop(0, (N_MODES * DIM) // L, exp_body, 0)

        cvec = c_v[...]
        lane_iota = lax.iota(jnp.int32, L)

        def chunk_body(ci, _):
            sbase = wid * per_w + ci * CH
            pltpu.sync_copy(idx_hbm.at[pl.ds(sbase, CH)], idx_v)
            pltpu.sync_copy(eps_hbm.at[pl.ds(sbase, CH), :], eps_v)

            def group_body(g, _):
                trow_vec = idx_v[pl.ds(g * L, L)] * DIM
                sums = jnp.zeros((L,), jnp.float32)
                for u in range(L):
                    s = g * L + u
                    trow = trow_vec[u]
                    acc = jnp.zeros((L,), jnp.float32)
                    for b in range(DIM // L):
                        ev = eps_v[s, pl.ds(b * L, L)]
                        sv = scale_v[pl.ds(trow + b * L, L)]
                        lv = loc_v[pl.ds(trow + b * L, L)]
                        z_v[s, pl.ds(b * L, L)] = ev * sv + lv
                        acc = acc + ev * ev
                    sums = jnp.where(lane_iota == u, jnp.sum(acc), sums)
                lp_v[pl.ds(g * L, L)] = cvec - 0.5 * sums
                return 0
            lax.fori_loop(0, CH // L, group_body, 0)

            pltpu.sync_copy(z_v, z_hbm.at[pl.ds(sbase, CH), :])
            pltpu.sync_copy(lp_v, lp_hbm.at[pl.ds(sbase, CH)])
            return 0
        lax.fori_loop(0, nch, chunk_body, 0)

    return k


def kernel(eps, loc, log_scale, weight_scores, mode_ind, num_samples):
    n = eps.shape[0]
    log_w = jax.nn.log_softmax(weight_scores, axis=1)
    per_mode = log_w - jnp.sum(log_scale, axis=2)
    c = (-0.5 * DIM * math.log(2.0 * math.pi)
         + jax.scipy.special.logsumexp(per_mode, axis=1))
    c_arr = jnp.broadcast_to(c.astype(jnp.float32), (L,))

    z, log_p = _sc_kernel(n)(
        log_scale.reshape(-1).astype(jnp.float32),
        loc.reshape(-1).astype(jnp.float32),
        c_arr,
        mode_ind.astype(jnp.int32),
        eps,
    )
    return z, log_p


# speedup vs baseline: 6.2812x; 1.7098x over previous
import functools
import math

import jax
import jax.numpy as jnp
from jax import lax
from jax.experimental import pallas as pl
from jax.experimental.pallas import tpu as pltpu
from jax.experimental.pallas import tpu_sc as plsc

N_MODES = 64
DIM = 64
NC = 2
NS = 16
NW = NC * NS
L = 16
CH = 512


def _sc_kernel(n):
    mesh = plsc.VectorSubcoreMesh(core_axis_name="c", subcore_axis_name="s")
    per_w = n // NW
    nch = per_w // CH

    @functools.partial(
        pl.kernel,
        mesh=mesh,
        compiler_params=pltpu.CompilerParams(needs_layout_passes=False),
        out_type=[
            jax.ShapeDtypeStruct((DIM, n), jnp.float32),
            jax.ShapeDtypeStruct((n,), jnp.float32),
        ],
        scratch_types=[
            pltpu.VMEM((N_MODES * DIM,), jnp.float32),
            pltpu.VMEM((N_MODES * DIM,), jnp.float32),
            pltpu.VMEM((L,), jnp.float32),
            pltpu.VMEM((CH,), jnp.int32),
            pltpu.VMEM((DIM, CH), jnp.float32),
            pltpu.VMEM((DIM, CH), jnp.float32),
            pltpu.VMEM((CH,), jnp.float32),
        ],
    )
    def k(ls_hbm, loc_hbm, c_hbm, idx_hbm, eps_hbm,
          z_hbm, lp_hbm,
          scale_v, loc_v, c_v, idx_v, eps_v, z_v, lp_v):
        wid = lax.axis_index("s") * NC + lax.axis_index("c")

        pltpu.sync_copy(ls_hbm, scale_v)
        pltpu.sync_copy(loc_hbm, loc_v)
        pltpu.sync_copy(c_hbm, c_v)

        def exp_body(i, _):
            scale_v[pl.ds(i * L, L)] = jnp.exp(scale_v[pl.ds(i * L, L)])
            return 0
        lax.fori_loop(0, (N_MODES * DIM) // L, exp_body, 0)

        cvec = c_v[...]

        def chunk_body(ci, _):
            sbase = wid * per_w + ci * CH
            pltpu.sync_copy(idx_hbm.at[pl.ds(sbase, CH)], idx_v)
            pltpu.sync_copy(eps_hbm.at[:, pl.ds(sbase, CH)], eps_v)

            def group_body(g, _):
                mvec = idx_v[pl.ds(g * L, L)]
                acc0 = jnp.zeros((L,), jnp.float32)
                acc1 = jnp.zeros((L,), jnp.float32)
                for d in range(DIM):
                    tidx = mvec + (d * N_MODES)
                    sv = plsc.load_gather(scale_v, [tidx])
                    lv = plsc.load_gather(loc_v, [tidx])
                    ev = eps_v[d, pl.ds(g * L, L)]
                    z_v[d, pl.ds(g * L, L)] = ev * sv + lv
                    if d % 2 == 0:
                        acc0 = acc0 + ev * ev
                    else:
                        acc1 = acc1 + ev * ev
                lp_v[pl.ds(g * L, L)] = cvec - 0.5 * (acc0 + acc1)
                return 0
            lax.fori_loop(0, CH // L, group_body, 0)

            pltpu.sync_copy(z_v, z_hbm.at[:, pl.ds(sbase, CH)])
            pltpu.sync_copy(lp_v, lp_hbm.at[pl.ds(sbase, CH)])
            return 0
        lax.fori_loop(0, nch, chunk_body, 0)

    return k


def kernel(eps, loc, log_scale, weight_scores, mode_ind, num_samples):
    n = eps.shape[0]
    log_w = jax.nn.log_softmax(weight_scores, axis=1)
    per_mode = log_w - jnp.sum(log_scale, axis=2)
    c = (-0.5 * DIM * math.log(2.0 * math.pi)
         + jax.scipy.special.logsumexp(per_mode, axis=1))
    c_arr = jnp.broadcast_to(c.astype(jnp.float32), (L,))

    ls_cm = jnp.swapaxes(log_scale[0], 0, 1).reshape(-1)
    loc_cm = jnp.swapaxes(loc[0], 0, 1).reshape(-1)

    z_t, log_p = _sc_kernel(n)(
        ls_cm.astype(jnp.float32),
        loc_cm.astype(jnp.float32),
        c_arr,
        mode_ind.astype(jnp.int32),
        eps.T,
    )
    return z_t.T, log_p


# speedup vs baseline: 8.6241x; 1.3730x over previous
import functools
import math

import jax
import jax.numpy as jnp
from jax import lax
from jax.experimental import pallas as pl
from jax.experimental.pallas import tpu as pltpu
from jax.experimental.pallas import tpu_sc as plsc

N_MODES = 64
DIM = 64
NC = 2
NS = 16
NW = NC * NS
L = 16
CH = 512


def _sc_kernel(n):
    mesh = plsc.VectorSubcoreMesh(core_axis_name="c", subcore_axis_name="s")
    per_w = n // NW
    nch = per_w // CH

    @functools.partial(
        pl.kernel,
        mesh=mesh,
        compiler_params=pltpu.CompilerParams(needs_layout_passes=False),
        out_type=[
            jax.ShapeDtypeStruct((DIM, n), jnp.float32),
            jax.ShapeDtypeStruct((n,), jnp.float32),
        ],
        scratch_types=[
            pltpu.VMEM((N_MODES * DIM,), jnp.float32),
            pltpu.VMEM((N_MODES * DIM,), jnp.float32),
            pltpu.VMEM((L,), jnp.float32),
            pltpu.VMEM((CH,), jnp.int32),
            pltpu.VMEM((DIM, CH), jnp.float32),
            pltpu.VMEM((DIM, CH), jnp.float32),
            pltpu.VMEM((CH,), jnp.float32),
        ],
    )
    def k(ls_hbm, loc_hbm, c_hbm, idx_hbm, eps_hbm,
          z_hbm, lp_hbm,
          scale_v, loc_v, c_v, idx_v, eps_v, z_v, lp_v):
        wid = lax.axis_index("s") * NC + lax.axis_index("c")

        pltpu.sync_copy(ls_hbm, scale_v)
        pltpu.sync_copy(loc_hbm, loc_v)
        pltpu.sync_copy(c_hbm, c_v)

        def exp_body(i, _):
            scale_v[pl.ds(i * L, L)] = jnp.exp(scale_v[pl.ds(i * L, L)])
            return 0
        lax.fori_loop(0, (N_MODES * DIM) // L, exp_body, 0)

        cvec = c_v[...]

        def chunk_body(ci, _):
            sbase = wid * per_w + ci * CH
            pltpu.sync_copy(idx_hbm.at[pl.ds(sbase, CH)], idx_v)
            pltpu.sync_copy(eps_hbm.at[:, pl.ds(sbase, CH)], eps_v)

            @plsc.parallel_loop(0, CH // L, unroll=2)
            def group_body(g):
                mvec = idx_v[pl.ds(g * L, L)]
                acc0 = jnp.zeros((L,), jnp.float32)
                acc1 = jnp.zeros((L,), jnp.float32)
                for d in range(DIM):
                    tidx = mvec + (d * N_MODES)
                    sv = plsc.load_gather(scale_v, [tidx])
                    lv = plsc.load_gather(loc_v, [tidx])
                    ev = eps_v[d, pl.ds(g * L, L)]
                    z_v[d, pl.ds(g * L, L)] = ev * sv + lv
                    if d % 2 == 0:
                        acc0 = acc0 + ev * ev
                    else:
                        acc1 = acc1 + ev * ev
                lp_v[pl.ds(g * L, L)] = cvec - 0.5 * (acc0 + acc1)

            pltpu.sync_copy(z_v, z_hbm.at[:, pl.ds(sbase, CH)])
            pltpu.sync_copy(lp_v, lp_hbm.at[pl.ds(sbase, CH)])
            return 0
        lax.fori_loop(0, nch, chunk_body, 0)

    return k


def kernel(eps, loc, log_scale, weight_scores, mode_ind, num_samples):
    n = eps.shape[0]
    log_w = jax.nn.log_softmax(weight_scores, axis=1)
    per_mode = log_w - jnp.sum(log_scale, axis=2)
    c = (-0.5 * DIM * math.log(2.0 * math.pi)
         + jax.scipy.special.logsumexp(per_mode, axis=1))
    c_arr = jnp.broadcast_to(c.astype(jnp.float32), (L,))

    ls_cm = jnp.swapaxes(log_scale[0], 0, 1).reshape(-1)
    loc_cm = jnp.swapaxes(loc[0], 0, 1).reshape(-1)

    z_t, log_p = _sc_kernel(n)(
        ls_cm.astype(jnp.float32),
        loc_cm.astype(jnp.float32),
        c_arr,
        mode_ind.astype(jnp.int32),
        eps.T,
    )
    return z_t.T, log_p
